# baseline (speedup 1.0000x reference)
import functools

import jax
import jax.numpy as jnp
from jax import lax
from jax.experimental import pallas as pl
from jax.experimental.pallas import tpu as pltpu
from jax.experimental.pallas import tpu_sc as plsc

P, C, D = 5, 100000, 64
B, T = 4096, 200
N = B * T

_ROWS = 2048
_NFG = -(-C // _ROWS)
_BG_ROW = _NFG * _ROWS
_TABLE_ROWS = (_NFG + 1) * _ROWS


def _table_body(fg_ref, bg_ref, out_ref):
    j = pl.program_id(0)

    @pl.when(j < _NFG)
    def _fg():
        x = fg_ref[...]
        m = jnp.sum(x, axis=0) * (1.0 / P)
        norm = jnp.sqrt(jnp.sum(m * m, axis=0, keepdims=True))
        normed = m / jnp.maximum(norm, 1e-5)
        out_ref[...] = normed.T

    @pl.when(j == _NFG)
    def _bg():
        b = bg_ref[...]
        norm = jnp.sqrt(jnp.sum(b * b, axis=1, keepdims=True))
        out_ref[...] = jnp.broadcast_to(b / jnp.maximum(norm, 1e-5), (_ROWS, D))


def _build_table(fg, bg):
    fg_t = jnp.transpose(fg, (0, 2, 1))
    return pl.pallas_call(
        _table_body,
        grid=(_NFG + 1,),
        in_specs=[
            pl.BlockSpec((P, D, _ROWS), lambda j: (0, 0, jnp.minimum(j, _NFG - 1))),
            pl.BlockSpec((1, D), lambda j: (0, 0)),
        ],
        out_specs=pl.BlockSpec((_ROWS, D), lambda j: (j, 0)),
        out_shape=jax.ShapeDtypeStruct((_TABLE_ROWS, D), jnp.float32),
    )(fg_t, bg)


def _remap_body(t_ref, out_ref):
    t = t_ref[...]
    r = jnp.where(t == 0, _BG_ROW, t - 1)
    out_ref[...] = jnp.transpose(r.reshape(T, 2, 512), (0, 2, 1)).reshape(T, 1024)


def _remap_indices(transcripts):
    t_t = jnp.transpose(transcripts.astype(jnp.int32))
    return pl.pallas_call(
        _remap_body,
        grid=(B // 1024,),
        in_specs=[pl.BlockSpec((T, 1024), lambda j: (0, j))],
        out_specs=pl.BlockSpec((T, 1024), lambda j: (0, j)),
        out_shape=jax.ShapeDtypeStruct((T, B), jnp.int32),
    )(t_t)


_NC, _NS = 2, 16
_NW = _NC * _NS
_BPW = B // _NW
_K = 8
_NSUPER = T // _K


def _gather_body(table_hbm, idx_hbm, out_hbm, idx_v, rows_v, gsem, ssem):
    wid = lax.axis_index("s") * _NC + lax.axis_index("c")
    b0 = wid * _BPW
    pltpu.sync_copy(idx_hbm.at[:, pl.ds(b0, _BPW)], idx_v)

    def superstep(s, carry):
        cps = []
        for i in range(_K):
            cps.append(
                pltpu.async_copy(
                    table_hbm.at[idx_v.at[s * _K + i]],
                    rows_v.at[i],
                    gsem,
                )
            )
        for cp in cps:
            cp.wait()
        sps = []
        for i in range(_K):
            sps.append(
                pltpu.async_copy(
                    rows_v.at[i],
                    out_hbm.at[s * _K + i, pl.ds(b0, _BPW)],
                    ssem,
                )
            )
        for sp in sps:
            sp.wait()
        return carry

    lax.fori_loop(0, _NSUPER, superstep, 0)


@functools.cache
def _make_gather():
    @functools.partial(
        pl.kernel,
        mesh=plsc.VectorSubcoreMesh(core_axis_name="c", subcore_axis_name="s"),
        out_type=jax.ShapeDtypeStruct((T, B, D), jnp.float32),
        compiler_params=pltpu.CompilerParams(use_tc_tiling_on_sc=False),
        scratch_types=[
            pltpu.VMEM((T, _BPW), jnp.int32),
            pltpu.VMEM((_K, _BPW, D), jnp.float32),
            pltpu.SemaphoreType.DMA,
            pltpu.SemaphoreType.DMA,
        ],
    )
    def _gather_rows(table_hbm, idx_hbm, out_hbm, idx_v, rows_v, gsem, ssem):
        _gather_body(table_hbm, idx_hbm, out_hbm, idx_v, rows_v, gsem, ssem)

    return _gather_rows


_FU = 512
_NU = (B // 2) // _FU


def _finish_body(x_ref, eye_ref, out_ref):
    x = x_ref[...]
    xt = lax.dot_general(
        x, eye_ref[...], (((0,), (0,)), ((), ())),
        preferred_element_type=jnp.float32,
    )
    out_ref[0, :, : _FU] = xt[:D]
    out_ref[0, :, _FU:] = xt[D:]


def _finish(pairs):
    eye = jnp.eye(_FU, dtype=jnp.float32)
    return pl.pallas_call(
        _finish_body,
        grid=(T * _NU,),
        in_specs=[
            pl.BlockSpec((_FU, 128), lambda j: (j, 0)),
            pl.BlockSpec((_FU, _FU), lambda j: (0, 0)),
        ],
        out_specs=pl.BlockSpec(
            (1, D, 2 * _FU), lambda j: (j // _NU, 0, jnp.remainder(j, _NU))
        ),
        out_shape=jax.ShapeDtypeStruct((T, D, B), jnp.float32),
    )(pairs, eye)


def kernel(transcripts, fg_action_embedding, bg_embedding):
    table = _build_table(fg_action_embedding, bg_embedding)
    idx = _remap_indices(transcripts)
    out = _make_gather()(table, idx)
    tdb = _finish(out.reshape(N // 2, 128))
    return jnp.transpose(tdb, (2, 0, 1))


# speedup vs baseline: 1.7733x; 1.7733x over previous
import functools

import jax
import jax.numpy as jnp
from jax import lax
from jax.experimental import pallas as pl
from jax.experimental.pallas import tpu as pltpu
from jax.experimental.pallas import tpu_sc as plsc

P, C, D = 5, 100000, 64
B, T = 4096, 200
N = B * T

_ROWS = 2048
_NFG = -(-C // _ROWS)
_BG_ROW = _NFG * _ROWS
_TABLE_ROWS = (_NFG + 1) * _ROWS


def _table_body(fg_ref, bg_ref, out_ref):
    j = pl.program_id(0)

    @pl.when(j < _NFG)
    def _fg():
        x = fg_ref[...]
        m = jnp.sum(x, axis=0) * (1.0 / P)
        norm = jnp.sqrt(jnp.sum(m * m, axis=0, keepdims=True))
        normed = (m / jnp.maximum(norm, 1e-5)).T
        out_ref[...] = jnp.concatenate(
            [normed, jnp.zeros((_ROWS, 128 - D), jnp.float32)], axis=1
        )

    @pl.when(j == _NFG)
    def _bg():
        b = bg_ref[...]
        norm = jnp.sqrt(jnp.sum(b * b, axis=1, keepdims=True))
        normed = jnp.concatenate(
            [b / jnp.maximum(norm, 1e-5), jnp.zeros((1, 128 - D), jnp.float32)],
            axis=1,
        )
        out_ref[...] = jnp.broadcast_to(normed, (_ROWS, 128))


def _build_table(fg, bg):
    fg_t = jnp.transpose(fg, (0, 2, 1))
    return pl.pallas_call(
        _table_body,
        grid=(_NFG + 1,),
        in_specs=[
            pl.BlockSpec((P, D, _ROWS), lambda j: (0, 0, jnp.minimum(j, _NFG - 1))),
            pl.BlockSpec((1, D), lambda j: (0, 0)),
        ],
        out_specs=pl.BlockSpec((_ROWS, 128), lambda j: (j, 0)),
        out_shape=jax.ShapeDtypeStruct((_TABLE_ROWS, 128), jnp.float32),
    )(fg_t, bg)


def _remap_body(t_ref, out_ref):
    t = t_ref[...]
    out_ref[...] = jnp.where(t == 0, _BG_ROW, t - 1)


def _remap_indices(transcripts):
    t_t = jnp.transpose(transcripts.astype(jnp.int32))
    return pl.pallas_call(
        _remap_body,
        grid=(8,),
        in_specs=[pl.BlockSpec((T, B // 8), lambda j: (0, j))],
        out_specs=pl.BlockSpec((T, B // 8), lambda j: (0, j)),
        out_shape=jax.ShapeDtypeStruct((T, B), jnp.int32),
    )(t_t)


_NC, _NS = 2, 16
_NW = _NC * _NS
_BPW = B // _NW
_K = 4
_NSUPER = T // _K


def _gather_body(table_hbm, idx_hbm, out_hbm, idx_v, rows_v, gsem, ssem):
    wid = lax.axis_index("s") * _NC + lax.axis_index("c")
    b0 = wid * _BPW
    pltpu.sync_copy(idx_hbm.at[:, pl.ds(b0, _BPW)], idx_v)

    def superstep(s, carry):
        cps = []
        for i in range(_K):
            cps.append(
                pltpu.async_copy(
                    table_hbm.at[idx_v.at[s * _K + i]],
                    rows_v.at[i],
                    gsem,
                )
            )
        for cp in cps:
            cp.wait()
        sps = []
        for i in range(_K):
            sps.append(
                pltpu.async_copy(
                    rows_v.at[i],
                    out_hbm.at[s * _K + i, pl.ds(b0, _BPW)],
                    ssem,
                )
            )
        for sp in sps:
            sp.wait()
        return carry

    lax.fori_loop(0, _NSUPER, superstep, 0)


@functools.cache
def _make_gather():
    @functools.partial(
        pl.kernel,
        mesh=plsc.VectorSubcoreMesh(core_axis_name="c", subcore_axis_name="s"),
        out_type=jax.ShapeDtypeStruct((T, B, 128), jnp.float32),
        compiler_params=pltpu.CompilerParams(use_tc_tiling_on_sc=False),
        scratch_types=[
            pltpu.VMEM((T, _BPW), jnp.int32),
            pltpu.VMEM((_K, _BPW, 128), jnp.float32),
            pltpu.SemaphoreType.DMA,
            pltpu.SemaphoreType.DMA,
        ],
    )
    def _gather_rows(table_hbm, idx_hbm, out_hbm, idx_v, rows_v, gsem, ssem):
        _gather_body(table_hbm, idx_hbm, out_hbm, idx_v, rows_v, gsem, ssem)

    return _gather_rows


def kernel(transcripts, fg_action_embedding, bg_embedding):
    table = _build_table(fg_action_embedding, bg_embedding)
    idx = _remap_indices(transcripts)
    out = _make_gather()(table, idx)
    return jnp.transpose(out, (1, 0, 2))[:, :, :D]


# speedup vs baseline: 1.7958x; 1.0127x over previous
import functools

import jax
import jax.numpy as jnp
from jax import lax
from jax.experimental import pallas as pl
from jax.experimental.pallas import tpu as pltpu
from jax.experimental.pallas import tpu_sc as plsc

P, C, D = 5, 100000, 64
B, T = 4096, 200
N = B * T

_ROWS = 2048
_NFG = -(-C // _ROWS)
_BG_ROW = _NFG * _ROWS
_TABLE_ROWS = (_NFG + 1) * _ROWS


def _table_body(fg_ref, bg_ref, out_ref):
    j = pl.program_id(0)

    @pl.when(j < _NFG)
    def _fg():
        x = fg_ref[...]
        m = jnp.sum(x, axis=0) * (1.0 / P)
        norm = jnp.sqrt(jnp.sum(m * m, axis=0, keepdims=True))
        normed = (m / jnp.maximum(norm, 1e-5)).T
        out_ref[...] = jnp.concatenate(
            [normed, jnp.zeros((_ROWS, 128 - D), jnp.float32)], axis=1
        )

    @pl.when(j == _NFG)
    def _bg():
        b = bg_ref[...]
        norm = jnp.sqrt(jnp.sum(b * b, axis=1, keepdims=True))
        normed = jnp.concatenate(
            [b / jnp.maximum(norm, 1e-5), jnp.zeros((1, 128 - D), jnp.float32)],
            axis=1,
        )
        out_ref[...] = jnp.broadcast_to(normed, (_ROWS, 128))


def _build_table(fg, bg):
    fg_t = jnp.transpose(fg, (0, 2, 1))
    return pl.pallas_call(
        _table_body,
        grid=(_NFG + 1,),
        in_specs=[
            pl.BlockSpec((P, D, _ROWS), lambda j: (0, 0, jnp.minimum(j, _NFG - 1))),
            pl.BlockSpec((1, D), lambda j: (0, 0)),
        ],
        out_specs=pl.BlockSpec((_ROWS, 128), lambda j: (j, 0)),
        out_shape=jax.ShapeDtypeStruct((_TABLE_ROWS, 128), jnp.float32),
    )(fg_t, bg)


def _remap_body(t_ref, out_ref):
    t = t_ref[...]
    out_ref[...] = jnp.where(t == 0, _BG_ROW, t - 1)


def _remap_indices(transcripts):
    t_t = jnp.transpose(transcripts.astype(jnp.int32))
    return pl.pallas_call(
        _remap_body,
        grid=(8,),
        in_specs=[pl.BlockSpec((T, B // 8), lambda j: (0, j))],
        out_specs=pl.BlockSpec((T, B // 8), lambda j: (0, j)),
        out_shape=jax.ShapeDtypeStruct((T, B), jnp.int32),
    )(t_t)


_NC, _NS = 2, 16
_NW = _NC * _NS
_BPW = B // _NW
_K = 5
_NSUPER = T // _K


def _gather_body(table_hbm, idx_hbm, out_hbm, idx_v, rows_v, gsem, ssem):
    wid = lax.axis_index("s") * _NC + lax.axis_index("c")
    b0 = wid * _BPW
    pltpu.sync_copy(idx_hbm.at[:, pl.ds(b0, _BPW)], idx_v)

    def superstep(s, carry):
        cps = []
        for i in range(_K):
            cps.append(
                pltpu.async_copy(
                    table_hbm.at[idx_v.at[s * _K + i]],
                    rows_v.at[i],
                    gsem,
                )
            )
        for cp in cps:
            cp.wait()
        sps = []
        for i in range(_K):
            sps.append(
                pltpu.async_copy(
                    rows_v.at[i],
                    out_hbm.at[s * _K + i, pl.ds(b0, _BPW)],
                    ssem,
                )
            )
        for sp in sps:
            sp.wait()
        return carry

    lax.fori_loop(0, _NSUPER, superstep, 0)


@functools.cache
def _make_gather():
    @functools.partial(
        pl.kernel,
        mesh=plsc.VectorSubcoreMesh(core_axis_name="c", subcore_axis_name="s"),
        out_type=jax.ShapeDtypeStruct((T, B, 128), jnp.float32),
        compiler_params=pltpu.CompilerParams(use_tc_tiling_on_sc=False),
        scratch_types=[
            pltpu.VMEM((T, _BPW), jnp.int32),
            pltpu.VMEM((_K, _BPW, 128), jnp.float32),
            pltpu.SemaphoreType.DMA,
            pltpu.SemaphoreType.DMA,
        ],
    )
    def _gather_rows(table_hbm, idx_hbm, out_hbm, idx_v, rows_v, gsem, ssem):
        _gather_body(table_hbm, idx_hbm, out_hbm, idx_v, rows_v, gsem, ssem)

    return _gather_rows


def kernel(transcripts, fg_action_embedding, bg_embedding):
    table = _build_table(fg_action_embedding, bg_embedding)
    idx = _remap_indices(transcripts)
    out = _make_gather()(table, idx)
    return jnp.transpose(out, (1, 0, 2))[:, :, :D]


# speedup vs baseline: 2.2858x; 1.2728x over previous
import functools

import jax
import jax.numpy as jnp
from jax import lax
from jax.experimental import pallas as pl
from jax.experimental.pallas import tpu as pltpu
from jax.experimental.pallas import tpu_sc as plsc

P, C, D = 5, 100000, 64
B, T = 4096, 200
N = B * T

_ROWS = 2048
_NFG = -(-C // _ROWS)
_BG_ROW = _NFG * _ROWS
_TABLE_ROWS = (_NFG + 1) * _ROWS


def _table_body(fg_ref, bg_ref, out_ref):
    j = pl.program_id(0)

    @pl.when(j < _NFG)
    def _fg():
        x = fg_ref[...]
        m = jnp.sum(x, axis=0) * (1.0 / P)
        norm = jnp.sqrt(jnp.sum(m * m, axis=0, keepdims=True))
        out_ref[...] = (m / jnp.maximum(norm, 1e-5)).T

    @pl.when(j == _NFG)
    def _bg():
        b = bg_ref[...]
        norm = jnp.sqrt(jnp.sum(b * b, axis=1, keepdims=True))
        out_ref[...] = jnp.broadcast_to(b / jnp.maximum(norm, 1e-5), (_ROWS, D))


def _build_table(fg, bg):
    fg_t = jnp.transpose(fg, (0, 2, 1))
    return pl.pallas_call(
        _table_body,
        grid=(_NFG + 1,),
        in_specs=[
            pl.BlockSpec((P, D, _ROWS), lambda j: (0, 0, jnp.minimum(j, _NFG - 1))),
            pl.BlockSpec((1, D), lambda j: (0, 0)),
        ],
        out_specs=pl.BlockSpec((_ROWS, D), lambda j: (j, 0)),
        out_shape=jax.ShapeDtypeStruct((_TABLE_ROWS, D), jnp.float32),
    )(fg_t, bg)


def _remap_body(t_ref, out_ref):
    t = t_ref[...]
    out_ref[...] = jnp.where(t == 0, _BG_ROW, t - 1)


def _remap_indices(transcripts):
    t_t = jnp.transpose(transcripts.astype(jnp.int32))
    return pl.pallas_call(
        _remap_body,
        grid=(8,),
        in_specs=[pl.BlockSpec((T, B // 8), lambda j: (0, j))],
        out_specs=pl.BlockSpec((T, B // 8), lambda j: (0, j)),
        out_shape=jax.ShapeDtypeStruct((T, B), jnp.int32),
    )(t_t)


_NC, _NS = 2, 16
_NW = _NC * _NS
_BPW = B // _NW
_K = 10
_NSUPER = T // _K


def _gather_body(table_hbm, idx_hbm, out_hbm, idx_v, rows_v, gsem, ssem):
    wid = lax.axis_index("s") * _NC + lax.axis_index("c")
    b0 = wid * _BPW
    pltpu.sync_copy(idx_hbm.at[:, pl.ds(b0, _BPW)], idx_v)

    def superstep(s, carry):
        cps = []
        for i in range(_K):
            cps.append(
                pltpu.async_copy(
                    table_hbm.at[idx_v.at[s * _K + i]],
                    rows_v.at[i],
                    gsem,
                )
            )
        for cp in cps:
            cp.wait()
        sps = []
        for i in range(_K):
            sps.append(
                pltpu.async_copy(
                    rows_v.at[i],
                    out_hbm.at[s * _K + i, pl.ds(b0, _BPW), pl.ds(0, D)],
                    ssem,
                )
            )
        for sp in sps:
            sp.wait()
        return carry

    lax.fori_loop(0, _NSUPER, superstep, 0)


@functools.cache
def _make_gather():
    @functools.partial(
        pl.kernel,
        mesh=plsc.VectorSubcoreMesh(core_axis_name="c", subcore_axis_name="s"),
        out_type=jax.ShapeDtypeStruct((T, B, 128), jnp.float32),
        compiler_params=pltpu.CompilerParams(use_tc_tiling_on_sc=False),
        scratch_types=[
            pltpu.VMEM((T, _BPW), jnp.int32),
            pltpu.VMEM((_K, _BPW, D), jnp.float32),
            pltpu.SemaphoreType.DMA,
            pltpu.SemaphoreType.DMA,
        ],
    )
    def _gather_rows(table_hbm, idx_hbm, out_hbm, idx_v, rows_v, gsem, ssem):
        _gather_body(table_hbm, idx_hbm, out_hbm, idx_v, rows_v, gsem, ssem)

    return _gather_rows


def kernel(transcripts, fg_action_embedding, bg_embedding):
    table = _build_table(fg_action_embedding, bg_embedding)
    idx = _remap_indices(transcripts)
    out = _make_gather()(table, idx)
    return jnp.transpose(out, (1, 0, 2))[:, :, :D]
